# baseline (speedup 1.0000x reference)
import functools

import jax
import jax.numpy as jnp
from jax.experimental import pallas as pl
from jax.experimental.pallas import tpu as pltpu

N_LEVELS = 8
TILE = 2048
CHAINS = 8


def _rq_tile_kernel(x_ref, embeds_ref, cs_ref, inds_ref, qx_ref, nsmall_ref,
                    loss_ref, ecat_ref, e2_ref, *, batch):
    i = pl.program_id(0)
    d = x_ref.shape[1]
    k = embeds_ref.shape[2]

    @pl.when(i == 0)
    def _init():
        nsmall_ref[...] = jnp.sum(
            (cs_ref[...] < 1.0).astype(jnp.float32)).reshape(1, 1)
        loss_ref[...] = jnp.zeros((1, 1), jnp.float32)
        for lvl in range(N_LEVELS):
            e = embeds_ref[lvl]
            et = e.T
            hi = et.astype(jnp.bfloat16)
            rem = et - hi.astype(jnp.float32)
            mid = rem.astype(jnp.bfloat16)
            lo = (rem - mid.astype(jnp.float32)).astype(jnp.bfloat16)
            ecat_ref[lvl, :, 0:d] = hi
            ecat_ref[lvl, :, d:2 * d] = mid
            ecat_ref[lvl, :, 2 * d:3 * d] = lo
            e2_ref[lvl, :] = jnp.sum(e * e, axis=0)

    h = x_ref.shape[0] // CHAINS
    halves = []
    for c in range(CHAINS):
        s = slice(c * h, (c + 1) * h)
        x = x_ref[s, :]
        halves.append({
            "s": s, "x": x, "residual": x, "qx": jnp.zeros_like(x),
            "rr": jnp.sum(x * x, axis=1),
        })
    loss_acc = jnp.float32(0.0)
    cols = jax.lax.broadcasted_iota(jnp.int32, (h, k), 1)
    for l in range(N_LEVELS):
        emb = embeds_ref[l]
        for st in halves:
            residual = st["residual"]
            scores = jax.lax.dot_general(
                residual, emb, (((1,), (0,)), ((), ())),
                preferred_element_type=jnp.float32)
            dist = (st["rr"][:, None] - 2.0 * scores) + e2_ref[l][None, :]
            ind = jnp.argmin(dist, axis=1)
            onehot = (cols == ind[:, None]).astype(jnp.bfloat16)
            qcat = jax.lax.dot_general(
                onehot, ecat_ref[l], (((1,), (0,)), ((), ())),
                preferred_element_type=jnp.float32)
            q = (qcat[:, 0:d] + qcat[:, d:2 * d]) + qcat[:, 2 * d:3 * d]
            t = q - residual
            q_ste = residual + t
            residual = residual - q_ste
            st["residual"] = residual
            st["qx"] = st["qx"] + q_ste
            loss_acc += jnp.sum(t * t)
            st["rr"] = jnp.sum(residual * residual, axis=1)
            inds_ref[st["s"], l] = ind
    for st in halves:
        qx_ref[st["s"], :] = st["qx"]
    loss_ref[...] += (loss_acc / jnp.float32(batch * d)).reshape(1, 1)


@jax.jit
def kernel(x, embeds, cluster_sizes):
    b, d = x.shape
    n_levels, _, k = embeds.shape
    grid = (b // TILE,)
    inds, qx, nsmall, loss = pl.pallas_call(
        functools.partial(_rq_tile_kernel, batch=b),
        grid=grid,
        in_specs=[
            pl.BlockSpec((TILE, d), lambda i: (i, 0)),
            pl.BlockSpec((n_levels, d, k), lambda i: (0, 0, 0)),
            pl.BlockSpec((n_levels, k), lambda i: (0, 0)),
        ],
        out_specs=[
            pl.BlockSpec((TILE, n_levels), lambda i: (i, 0)),
            pl.BlockSpec((TILE, d), lambda i: (i, 0)),
            pl.BlockSpec((1, 1), lambda i: (0, 0)),
            pl.BlockSpec((1, 1), lambda i: (0, 0)),
        ],
        out_shape=[
            jax.ShapeDtypeStruct((b, n_levels), jnp.int32),
            jax.ShapeDtypeStruct((b, d), jnp.float32),
            jax.ShapeDtypeStruct((1, 1), jnp.float32),
            jax.ShapeDtypeStruct((1, 1), jnp.float32),
        ],
        scratch_shapes=[
            pltpu.VMEM((n_levels, k, 3 * d), jnp.bfloat16),
            pltpu.VMEM((n_levels, k), jnp.float32),
        ],
        compiler_params=pltpu.CompilerParams(
            dimension_semantics=("arbitrary",)),
    )(x, embeds, cluster_sizes)
    return (inds.astype(jnp.int64), qx, nsmall.reshape(()), loss.reshape(()))


# speedup vs baseline: 1.1813x; 1.1813x over previous
import functools

import jax
import jax.numpy as jnp
from jax.experimental import pallas as pl
from jax.experimental.pallas import tpu as pltpu

N_LEVELS = 8
TILE = 1024
CHAINS = 4


def _rq_tile_kernel(x_ref, embeds_ref, cs_ref, inds_ref, qx_ref, nsmall_ref,
                    loss_ref, ecat_ref, e2_ref, *, batch):
    i = pl.program_id(0)
    d = x_ref.shape[1]
    k = embeds_ref.shape[2]

    @pl.when(i == 0)
    def _init():
        nsmall_ref[...] = jnp.sum(
            (cs_ref[...] < 1.0).astype(jnp.float32)).reshape(1, 1)
        loss_ref[...] = jnp.zeros((1, 1), jnp.float32)
        for lvl in range(N_LEVELS):
            e = embeds_ref[lvl]
            et = e.T
            hi = et.astype(jnp.bfloat16)
            rem = et - hi.astype(jnp.float32)
            mid = rem.astype(jnp.bfloat16)
            lo = (rem - mid.astype(jnp.float32)).astype(jnp.bfloat16)
            ecat_ref[lvl, :, 0:d] = hi
            ecat_ref[lvl, :, d:2 * d] = mid
            ecat_ref[lvl, :, 2 * d:3 * d] = lo
            e2_ref[lvl, :] = jnp.sum(e * e, axis=0)

    h = x_ref.shape[0] // CHAINS
    halves = []
    for c in range(CHAINS):
        s = slice(c * h, (c + 1) * h)
        x = x_ref[s, :]
        halves.append({
            "s": s, "x": x, "residual": x, "qx": jnp.zeros_like(x),
            "rr": jnp.sum(x * x, axis=1),
        })
    loss_acc = jnp.float32(0.0)
    cols = jax.lax.broadcasted_iota(jnp.int32, (h, k), 1)
    for l in range(N_LEVELS):
        emb = embeds_ref[l]
        for st in halves:
            residual = st["residual"]
            scores = jax.lax.dot_general(
                residual, emb, (((1,), (0,)), ((), ())),
                preferred_element_type=jnp.float32)
            dist = (st["rr"][:, None] - 2.0 * scores) + e2_ref[l][None, :]
            ind = jnp.argmin(dist, axis=1)
            onehot = (cols == ind[:, None]).astype(jnp.bfloat16)
            qcat = jax.lax.dot_general(
                onehot, ecat_ref[l], (((1,), (0,)), ((), ())),
                preferred_element_type=jnp.float32)
            q = (qcat[:, 0:d] + qcat[:, d:2 * d]) + qcat[:, 2 * d:3 * d]
            t = q - residual
            q_ste = residual + t
            residual = residual - q_ste
            st["residual"] = residual
            st["qx"] = st["qx"] + q_ste
            loss_acc += jnp.sum(t * t)
            st["rr"] = jnp.sum(residual * residual, axis=1)
            inds_ref[st["s"], l] = ind
    for st in halves:
        qx_ref[st["s"], :] = st["qx"]
    loss_ref[...] += (loss_acc / jnp.float32(batch * d)).reshape(1, 1)


@jax.jit
def kernel(x, embeds, cluster_sizes):
    b, d = x.shape
    n_levels, _, k = embeds.shape
    grid = (b // TILE,)
    inds, qx, nsmall, loss = pl.pallas_call(
        functools.partial(_rq_tile_kernel, batch=b),
        grid=grid,
        in_specs=[
            pl.BlockSpec((TILE, d), lambda i: (i, 0)),
            pl.BlockSpec((n_levels, d, k), lambda i: (0, 0, 0)),
            pl.BlockSpec((n_levels, k), lambda i: (0, 0)),
        ],
        out_specs=[
            pl.BlockSpec((TILE, n_levels), lambda i: (i, 0)),
            pl.BlockSpec((TILE, d), lambda i: (i, 0)),
            pl.BlockSpec((1, 1), lambda i: (0, 0)),
            pl.BlockSpec((1, 1), lambda i: (0, 0)),
        ],
        out_shape=[
            jax.ShapeDtypeStruct((b, n_levels), jnp.int32),
            jax.ShapeDtypeStruct((b, d), jnp.float32),
            jax.ShapeDtypeStruct((1, 1), jnp.float32),
            jax.ShapeDtypeStruct((1, 1), jnp.float32),
        ],
        scratch_shapes=[
            pltpu.VMEM((n_levels, k, 3 * d), jnp.bfloat16),
            pltpu.VMEM((n_levels, k), jnp.float32),
        ],
        compiler_params=pltpu.CompilerParams(
            dimension_semantics=("arbitrary",)),
    )(x, embeds, cluster_sizes)
    return (inds.astype(jnp.int64), qx, nsmall.reshape(()), loss.reshape(()))


# speedup vs baseline: 1.2130x; 1.0269x over previous
import functools

import jax
import jax.numpy as jnp
from jax.experimental import pallas as pl
from jax.experimental.pallas import tpu as pltpu

N_LEVELS = 8
TILE = 1024
CHAINS = 4


def _rq_tile_kernel(x_ref, embeds_ref, cs_ref, inds_ref, qx_ref, nsmall_ref,
                    loss_ref, ecat_ref, e2_ref, *, batch):
    i = pl.program_id(0)
    d = x_ref.shape[1]
    k = embeds_ref.shape[2]

    @pl.when(i == 0)
    def _init():
        nsmall_ref[...] = jnp.sum(
            (cs_ref[...] < 1.0).astype(jnp.float32)).reshape(1, 1)
        loss_ref[...] = jnp.zeros((1, 1), jnp.float32)
        for lvl in range(N_LEVELS):
            e = embeds_ref[lvl]
            et = e.T
            hi = et.astype(jnp.bfloat16)
            rem = et - hi.astype(jnp.float32)
            mid = rem.astype(jnp.bfloat16)
            lo = (rem - mid.astype(jnp.float32)).astype(jnp.bfloat16)
            ecat_ref[lvl, :, 0:d] = hi
            ecat_ref[lvl, :, d:2 * d] = mid
            ecat_ref[lvl, :, 2 * d:3 * d] = lo
            e2_ref[lvl, :] = jnp.sum(e * e, axis=0)

    h = x_ref.shape[0] // CHAINS
    halves = []
    for c in range(CHAINS):
        s = slice(c * h, (c + 1) * h)
        x = x_ref[s, :]
        halves.append({
            "s": s, "x": x, "residual": x, "qx": jnp.zeros_like(x),
            "rr": jnp.sum(x * x, axis=1), "tsq": jnp.zeros_like(x),
            "inds": jnp.zeros((h, N_LEVELS), jnp.int32),
        })
    cols = jax.lax.broadcasted_iota(jnp.int32, (h, k), 1)
    lcols = jax.lax.broadcasted_iota(jnp.int32, (h, N_LEVELS), 1)
    for l in range(N_LEVELS):
        emb = embeds_ref[l]
        for st in halves:
            residual = st["residual"]
            scores = jax.lax.dot_general(
                residual, emb, (((1,), (0,)), ((), ())),
                preferred_element_type=jnp.float32)
            dist = (st["rr"][:, None] - 2.0 * scores) + e2_ref[l][None, :]
            ind = jnp.argmin(dist, axis=1)
            onehot = (cols == ind[:, None]).astype(jnp.bfloat16)
            qcat = jax.lax.dot_general(
                onehot, ecat_ref[l], (((1,), (0,)), ((), ())),
                preferred_element_type=jnp.float32)
            q = (qcat[:, 0:d] + qcat[:, d:2 * d]) + qcat[:, 2 * d:3 * d]
            t = q - residual
            q_ste = residual + t
            residual = residual - q_ste
            st["residual"] = residual
            st["qx"] = st["qx"] + q_ste
            st["tsq"] = st["tsq"] + t * t
            st["rr"] = jnp.sum(residual * residual, axis=1)
            st["inds"] = jnp.where(lcols == l, ind[:, None], st["inds"])
    loss_acc = jnp.float32(0.0)
    for st in halves:
        qx_ref[st["s"], :] = st["qx"]
        inds_ref[st["s"], :] = st["inds"]
        loss_acc += jnp.sum(st["tsq"])
    loss_ref[...] += (loss_acc / jnp.float32(batch * d)).reshape(1, 1)


@jax.jit
def kernel(x, embeds, cluster_sizes):
    b, d = x.shape
    n_levels, _, k = embeds.shape
    grid = (b // TILE,)
    inds, qx, nsmall, loss = pl.pallas_call(
        functools.partial(_rq_tile_kernel, batch=b),
        grid=grid,
        in_specs=[
            pl.BlockSpec((TILE, d), lambda i: (i, 0)),
            pl.BlockSpec((n_levels, d, k), lambda i: (0, 0, 0)),
            pl.BlockSpec((n_levels, k), lambda i: (0, 0)),
        ],
        out_specs=[
            pl.BlockSpec((TILE, n_levels), lambda i: (i, 0)),
            pl.BlockSpec((TILE, d), lambda i: (i, 0)),
            pl.BlockSpec((1, 1), lambda i: (0, 0)),
            pl.BlockSpec((1, 1), lambda i: (0, 0)),
        ],
        out_shape=[
            jax.ShapeDtypeStruct((b, n_levels), jnp.int32),
            jax.ShapeDtypeStruct((b, d), jnp.float32),
            jax.ShapeDtypeStruct((1, 1), jnp.float32),
            jax.ShapeDtypeStruct((1, 1), jnp.float32),
        ],
        scratch_shapes=[
            pltpu.VMEM((n_levels, k, 3 * d), jnp.bfloat16),
            pltpu.VMEM((n_levels, k), jnp.float32),
        ],
        compiler_params=pltpu.CompilerParams(
            dimension_semantics=("arbitrary",)),
    )(x, embeds, cluster_sizes)
    return (inds.astype(jnp.int64), qx, nsmall.reshape(()), loss.reshape(()))
